# baseline (speedup 1.0000x reference)
import jax
import jax.numpy as jnp
from jax import lax
from jax.experimental import pallas as pl
from jax.experimental.pallas import tpu as pltpu
from jax.experimental.pallas import tpu_sc as plsc

B = 16384
NUM = 13
NCAT = 26
V = 100000
D = 16
OUT = NUM + NCAT * D

NC = 2
NS = 16
L = 16
NW = NC * NS
RPW = (NCAT * D) // NW
Q = B // 4
CQ = Q // L
UF = 16


def _body(bn_hbm, ids_hbm, t2_hbm, sc_hbm, bi_hbm, out_hbm,
          idx_v, row_v, ga, gb, sc_v, bi_v,
          sem_s, sem_a, sem_b):
    wid = lax.axis_index("s") * NC + lax.axis_index("c")

    def start_stream(f, d):
        c1 = pltpu.make_async_copy(t2_hbm.at[f, d], row_v, sem_s)
        c1.start()
        return c1

    r0 = wid * RPW
    cps = start_stream(r0 // D, r0 % D)
    pltpu.sync_copy(sc_hbm, sc_v)
    pltpu.sync_copy(bi_hbm, bi_v)
    pltpu.sync_copy(ids_hbm.at[r0 // D], idx_v)
    lane0 = lax.iota(jnp.int32, L) == 0

    @pl.when(wid < NUM)
    def _num():
        widv = jnp.full((L,), 0, jnp.int32) + wid
        s16 = plsc.load_gather(sc_v, [widv])
        b16 = plsc.load_gather(bi_v, [widv])
        for q, buf in ((0, ga), (1, gb), (2, ga), (3, gb)):
            pltpu.sync_copy(bn_hbm.at[wid, pl.ds(q * Q, Q)], buf)

            def nchunk(k, _):
                x = buf[pl.ds(k * L, L)]
                buf[pl.ds(k * L, L)] = x * s16 + b16
                return 0
            lax.fori_loop(0, CQ, nchunk, 0)
            pltpu.sync_copy(buf, out_hbm.at[wid, pl.ds(q * Q, Q)])

    cpa = None
    cpb = None
    for j in range(RPW):
        r = r0 + j
        f = r // D
        c = r + NUM
        cps.wait()
        row_v[pl.ds(0, L)] = jnp.where(lane0, 0.0, row_v[pl.ds(0, L)])

        for q in range(4):
            buf = ga if q % 2 == 0 else gb
            if q % 2 == 0:
                if cpa is not None:
                    cpa.wait()
            else:
                if cpb is not None:
                    cpb.wait()

            def g4(k, _):
                for u in range(UF):
                    o = (k * UF + u) * L
                    iv = idx_v[pl.ds(q * Q + o, L)]
                    buf[pl.ds(o, L)] = plsc.load_gather(row_v, [iv])
                return 0
            lax.fori_loop(0, CQ // UF, g4, 0)

            sem = sem_a if q % 2 == 0 else sem_b
            cp = pltpu.make_async_copy(buf, out_hbm.at[c, pl.ds(q * Q, Q)],
                                       sem)
            cp.start()
            if q % 2 == 0:
                cpa = cp
            else:
                cpb = cp

        if j + 1 < RPW:
            r2 = r0 + j + 1
            f2 = r2 // D
            cps = start_stream(f2, r2 % D)

            @pl.when(f2 != f)
            def _restage():
                pltpu.sync_copy(ids_hbm.at[f2], idx_v)

    cpa.wait()
    cpb.wait()


@jax.jit
def kernel(batch_num, cat_ids, tables, num_mean, num_std):
    t2 = jnp.transpose(tables, (0, 2, 1))
    bn_t = batch_num.T
    scale = 1.0 / num_std.reshape(NUM)
    sc = jnp.pad(scale, (0, L - NUM))
    bi = jnp.pad(-num_mean.reshape(NUM) * scale, (0, L - NUM))

    mesh = plsc.VectorSubcoreMesh(core_axis_name="c", subcore_axis_name="s",
                                  num_cores=NC, num_subcores=NS)
    run = pl.kernel(
        _body,
        out_type=jax.ShapeDtypeStruct((OUT, B), jnp.float32),
        mesh=mesh,
        scratch_types=[
            pltpu.VMEM((B,), jnp.int32),
            pltpu.VMEM((V,), jnp.float32),
            pltpu.VMEM((Q,), jnp.float32),
            pltpu.VMEM((Q,), jnp.float32),
            pltpu.VMEM((L,), jnp.float32),
            pltpu.VMEM((L,), jnp.float32),
            pltpu.SemaphoreType.DMA,
            pltpu.SemaphoreType.DMA,
            pltpu.SemaphoreType.DMA,
        ],
        compiler_params=pltpu.CompilerParams(use_tc_tiling_on_sc=True,
                                             needs_layout_passes=False),
    )
    out = run(bn_t, cat_ids, t2, sc, bi)
    return out.T


# speedup vs baseline: 1.0151x; 1.0151x over previous
import jax
import jax.numpy as jnp
from jax import lax
from jax.experimental import pallas as pl
from jax.experimental.pallas import tpu as pltpu
from jax.experimental.pallas import tpu_sc as plsc

B = 16384
NUM = 13
NCAT = 26
V = 100000
D = 16
OUT = NUM + NCAT * D

NC = 2
NS = 16
L = 16
NW = NC * NS
RPW = (NCAT * D) // NW
Q = B // 4
CQ = Q // L
UF = 8


def _body(bn_hbm, ids_hbm, t2_hbm, sc_hbm, bi_hbm, out_hbm,
          idx_v, row_v, ga, gb, sc_v, bi_v,
          sem_s, sem_a, sem_b):
    wid = lax.axis_index("s") * NC + lax.axis_index("c")

    def start_stream(f, d):
        c1 = pltpu.make_async_copy(t2_hbm.at[f, d], row_v, sem_s)
        c1.start()
        return c1

    r0 = wid * RPW
    cps = start_stream(r0 // D, r0 % D)
    pltpu.sync_copy(sc_hbm, sc_v)
    pltpu.sync_copy(bi_hbm, bi_v)
    pltpu.sync_copy(ids_hbm.at[r0 // D], idx_v)
    lane0 = lax.iota(jnp.int32, L) == 0

    @pl.when(wid < NUM)
    def _num():
        widv = jnp.full((L,), 0, jnp.int32) + wid
        s16 = plsc.load_gather(sc_v, [widv])
        b16 = plsc.load_gather(bi_v, [widv])
        for q, buf in ((0, ga), (1, gb), (2, ga), (3, gb)):
            pltpu.sync_copy(bn_hbm.at[wid, pl.ds(q * Q, Q)], buf)

            def nchunk(k, _):
                x = buf[pl.ds(k * L, L)]
                buf[pl.ds(k * L, L)] = x * s16 + b16
                return 0
            lax.fori_loop(0, CQ, nchunk, 0)
            pltpu.sync_copy(buf, out_hbm.at[wid, pl.ds(q * Q, Q)])

    cpa = None
    cpb = None
    for j in range(RPW):
        r = r0 + j
        f = r // D
        c = r + NUM
        cps.wait()
        row_v[pl.ds(0, L)] = jnp.where(lane0, 0.0, row_v[pl.ds(0, L)])

        for q in range(4):
            buf = ga if q % 2 == 0 else gb
            if q % 2 == 0:
                if cpa is not None:
                    cpa.wait()
            else:
                if cpb is not None:
                    cpb.wait()

            def g4(k, _):
                for u in range(UF):
                    o = (k * UF + u) * L
                    iv = idx_v[pl.ds(q * Q + o, L)]
                    buf[pl.ds(o, L)] = plsc.load_gather(row_v, [iv])
                return 0
            lax.fori_loop(0, CQ // UF, g4, 0)

            sem = sem_a if q % 2 == 0 else sem_b
            cp = pltpu.make_async_copy(buf, out_hbm.at[c, pl.ds(q * Q, Q)],
                                       sem)
            cp.start()
            if q % 2 == 0:
                cpa = cp
            else:
                cpb = cp

        if j + 1 < RPW:
            r2 = r0 + j + 1
            f2 = r2 // D
            cps = start_stream(f2, r2 % D)

            @pl.when(f2 != f)
            def _restage():
                pltpu.sync_copy(ids_hbm.at[f2], idx_v)

    cpa.wait()
    cpb.wait()


@jax.jit
def kernel(batch_num, cat_ids, tables, num_mean, num_std):
    t2 = jnp.transpose(tables, (0, 2, 1))
    bn_t = batch_num.T
    scale = 1.0 / num_std.reshape(NUM)
    sc = jnp.pad(scale, (0, L - NUM))
    bi = jnp.pad(-num_mean.reshape(NUM) * scale, (0, L - NUM))

    mesh = plsc.VectorSubcoreMesh(core_axis_name="c", subcore_axis_name="s",
                                  num_cores=NC, num_subcores=NS)
    run = pl.kernel(
        _body,
        out_type=jax.ShapeDtypeStruct((OUT, B), jnp.float32),
        mesh=mesh,
        scratch_types=[
            pltpu.VMEM((B,), jnp.int32),
            pltpu.VMEM((V,), jnp.float32),
            pltpu.VMEM((Q,), jnp.float32),
            pltpu.VMEM((Q,), jnp.float32),
            pltpu.VMEM((L,), jnp.float32),
            pltpu.VMEM((L,), jnp.float32),
            pltpu.SemaphoreType.DMA,
            pltpu.SemaphoreType.DMA,
            pltpu.SemaphoreType.DMA,
        ],
        compiler_params=pltpu.CompilerParams(use_tc_tiling_on_sc=True,
                                             needs_layout_passes=False),
    )
    out = run(bn_t, cat_ids, t2, sc, bi)
    return out.T
